# baseline (speedup 1.0000x reference)
import functools

import jax
import jax.numpy as jnp
from jax import lax
from jax.experimental import pallas as pl
from jax.experimental.pallas import tpu as pltpu
from jax.experimental.pallas import tpu_sc as plsc

N = 10000
E = 320000
C_IN = 128
C_OUT = 128
KVOL = 27
EPS = 1e-5

NUM_CORES = 2
NUM_SUBCORES = 16
NUM_WORKERS = NUM_CORES * NUM_SUBCORES

CHUNK = 64
PHASE = 40
NPHASE = 4
ROWS_PER_W = PHASE * NPHASE
EPAD = NUM_WORKERS * ROWS_PER_W * CHUNK
NPAD = 10112
TPN = NPAD // NUM_SUBCORES

NB = 2000
NBC = N // NB


def _h_body(x_ref, w_ref, h_ref):
    h_ref[0] = jnp.dot(x_ref[...], w_ref[0], preferred_element_type=jnp.float32)


def _gidx_body(o_ref, s_ref, g_ref):
    g_ref[...] = o_ref[...] * N + s_ref[...]


def _bn_body(p_ref, g_ref, b_ref, o_ref):
    s = p_ref[0, :N] + p_ref[1, :N]
    m = jnp.mean(s, axis=0, keepdims=True)
    v = jnp.mean((s - m) ** 2, axis=0, keepdims=True)
    o_ref[...] = jnp.maximum((s - m) * lax.rsqrt(v + EPS) * g_ref[...] + b_ref[...],
                             0.0)


def _sc_body(h_hbm, gidx_hbm, dst_hbm, out_hbm,
             gidx_v, dst_v, rows0, rows1, rows2, rows3,
             acc, gs0, gs1, gs2, gs3, ss0, ss1, ss2, ss3):
    cid = lax.axis_index("c")
    sid = lax.axis_index("s")
    w = sid * NUM_CORES + cid
    rows = (rows0, rows1, rows2, rows3)
    gsem = (gs0, gs1, gs2, gs3)
    ssem = (ss0, ss1, ss2, ss3)

    def zrow(r, carry):
        for c in range(8):
            rows0[r, pl.ds(c * 16, 16)] = jnp.zeros((16,), jnp.float32)
        return carry

    lax.fori_loop(0, CHUNK, zrow, 0)
    for j in range((TPN + CHUNK - 1) // CHUNK):
        sz = min(CHUNK, TPN - j * CHUNK)
        pltpu.sync_copy(rows0.at[pl.ds(0, sz)],
                        acc.at[pl.ds(sid * TPN + j * CHUNK, sz)])

    plsc.subcore_barrier()

    def gather(i, b):
        h = CHUNK // 2
        pltpu.async_copy(h_hbm.at[gidx_v.at[i, pl.ds(0, h)]],
                         rows[b].at[pl.ds(0, h)], gsem[b])
        pltpu.async_copy(h_hbm.at[gidx_v.at[i, pl.ds(h, h)]],
                         rows[b].at[pl.ds(h, h)], gsem[b])

    def gwait(i, b):
        pltpu.make_async_copy(h_hbm.at[gidx_v.at[i]], rows[b], gsem[b]).wait()

    def sstart(i, b):
        pltpu.async_copy(rows[b], acc.at[dst_v.at[i]], ssem[b], add=True)

    def swait(i, b):
        pltpu.make_async_copy(rows[b], acc.at[dst_v.at[i]], ssem[b]).wait()

    def phase_body(ph, carry):
        pltpu.sync_copy(gidx_hbm.at[w, pl.ds(ph * PHASE, PHASE)], gidx_v)
        pltpu.sync_copy(dst_hbm.at[w, pl.ds(ph * PHASE, PHASE)], dst_v)
        gather(0, 0)
        gather(1, 1)
        gather(2, 2)

        def body(j, carry2):
            for b in range(4):
                i = 4 * j + b
                gwait(i, b)
                sstart(i, b)
                bn = (b + 3) % 4

                @pl.when(i >= 1)
                def _():
                    swait(i - 1, bn)

                @pl.when(i + 3 < PHASE)
                def _():
                    gather(i + 3, bn)

            return carry2

        lax.fori_loop(0, PHASE // 4, body, 0)
        swait(PHASE - 1, (PHASE - 1) % 4)
        return carry

    lax.fori_loop(0, NPHASE, phase_body, 0)
    plsc.subcore_barrier()

    pltpu.sync_copy(acc.at[pl.ds(sid * TPN, TPN)],
                    out_hbm.at[cid, pl.ds(sid * TPN, TPN)])


@functools.cache
def _sc_scatter():
    return pl.kernel(
        _sc_body,
        out_type=jax.ShapeDtypeStruct((NUM_CORES, NPAD, C_OUT), jnp.float32),
        mesh=plsc.VectorSubcoreMesh(core_axis_name="c", subcore_axis_name="s",
                                    num_cores=NUM_CORES,
                                    num_subcores=NUM_SUBCORES),
        scratch_types=[
            pltpu.VMEM((PHASE, CHUNK), jnp.int32),
            pltpu.VMEM((PHASE, CHUNK), jnp.int32),
            pltpu.VMEM((CHUNK, C_OUT), jnp.float32),
            pltpu.VMEM((CHUNK, C_OUT), jnp.float32),
            pltpu.VMEM((CHUNK, C_OUT), jnp.float32),
            pltpu.VMEM((CHUNK, C_OUT), jnp.float32),
            pltpu.VMEM_SHARED((NPAD, C_OUT), jnp.float32),
        ] + [pltpu.SemaphoreType.DMA] * 8,
    )


def kernel(x, edge_index, edge_offset, W, gamma, beta):
    src = edge_index[0]
    dst = edge_index[1]

    H = pl.pallas_call(
        _h_body,
        grid=(KVOL,),
        in_specs=[
            pl.BlockSpec((N, C_IN), lambda k: (0, 0)),
            pl.BlockSpec((1, C_IN, C_OUT), lambda k: (k, 0, 0)),
        ],
        out_specs=pl.BlockSpec((1, N, C_OUT), lambda k: (k, 0, 0)),
        out_shape=jax.ShapeDtypeStruct((KVOL, N, C_OUT), jnp.float32),
    )(x, W)
    H2 = H.reshape(KVOL * N, C_OUT)

    gidx = pl.pallas_call(
        _gidx_body,
        out_shape=jax.ShapeDtypeStruct((E // C_OUT, C_OUT), jnp.int32),
    )(edge_offset.reshape(E // C_OUT, C_OUT), src.reshape(E // C_OUT, C_OUT))

    npadE = EPAD - E
    gidx_p = jnp.concatenate(
        [gidx.reshape(E),
         (jnp.arange(npadE, dtype=jnp.int32) * 997) % (KVOL * N)])
    dst_p = jnp.concatenate(
        [dst, N + jnp.arange(npadE, dtype=jnp.int32) % (NPAD - N)])
    parts = _sc_scatter()(
        H2,
        gidx_p.reshape(NUM_WORKERS, ROWS_PER_W, CHUNK),
        dst_p.reshape(NUM_WORKERS, ROWS_PER_W, CHUNK),
    )

    out = pl.pallas_call(
        _bn_body,
        out_shape=jax.ShapeDtypeStruct((N, C_OUT), jnp.float32),
    )(parts, gamma.reshape(1, C_OUT), beta.reshape(1, C_OUT))
    return out


# speedup vs baseline: 1.0048x; 1.0048x over previous
import functools

import jax
import jax.numpy as jnp
from jax import lax
from jax.experimental import pallas as pl
from jax.experimental.pallas import tpu as pltpu
from jax.experimental.pallas import tpu_sc as plsc

N = 10000
E = 320000
C_IN = 128
C_OUT = 128
KVOL = 27
EPS = 1e-5

NUM_CORES = 2
NUM_SUBCORES = 16
NUM_WORKERS = NUM_CORES * NUM_SUBCORES

CHUNK = 80
PHASE = 32
NPHASE = 4
ROWS_PER_W = PHASE * NPHASE
EPAD = NUM_WORKERS * ROWS_PER_W * CHUNK
NPAD = 10112
TPN = NPAD // NUM_SUBCORES

NB = 2000
NBC = N // NB


def _h_body(x_ref, w_ref, h_ref):
    h_ref[0] = jnp.dot(x_ref[...], w_ref[0], preferred_element_type=jnp.float32)


def _gidx_body(o_ref, s_ref, g_ref):
    g_ref[...] = o_ref[...] * N + s_ref[...]


def _bn_body(p_ref, g_ref, b_ref, o_ref):
    s = p_ref[0, :N] + p_ref[1, :N]
    m = jnp.mean(s, axis=0, keepdims=True)
    v = jnp.mean((s - m) ** 2, axis=0, keepdims=True)
    o_ref[...] = jnp.maximum((s - m) * lax.rsqrt(v + EPS) * g_ref[...] + b_ref[...],
                             0.0)


def _sc_body(h_hbm, gidx_hbm, dst_hbm, out_hbm,
             gidx_v, dst_v, rows0, rows1, rows2, rows3,
             acc, gs0, gs1, gs2, gs3, ss0, ss1, ss2, ss3):
    cid = lax.axis_index("c")
    sid = lax.axis_index("s")
    w = sid * NUM_CORES + cid
    rows = (rows0, rows1, rows2, rows3)
    gsem = (gs0, gs1, gs2, gs3)
    ssem = (ss0, ss1, ss2, ss3)

    def zrow(r, carry):
        for c in range(8):
            rows0[r, pl.ds(c * 16, 16)] = jnp.zeros((16,), jnp.float32)
        return carry

    lax.fori_loop(0, CHUNK, zrow, 0)
    for j in range((TPN + CHUNK - 1) // CHUNK):
        sz = min(CHUNK, TPN - j * CHUNK)
        pltpu.sync_copy(rows0.at[pl.ds(0, sz)],
                        acc.at[pl.ds(sid * TPN + j * CHUNK, sz)])

    plsc.subcore_barrier()

    def gather(i, b):
        pltpu.async_copy(h_hbm.at[gidx_v.at[i]], rows[b], gsem[b])

    def gwait(i, b):
        pltpu.make_async_copy(h_hbm.at[gidx_v.at[i]], rows[b], gsem[b]).wait()

    def sstart(i, b):
        pltpu.async_copy(rows[b], acc.at[dst_v.at[i]], ssem[b], add=True)

    def swait(i, b):
        pltpu.make_async_copy(rows[b], acc.at[dst_v.at[i]], ssem[b]).wait()

    def phase_body(ph, carry):
        pltpu.sync_copy(gidx_hbm.at[w, pl.ds(ph * PHASE, PHASE)], gidx_v)
        pltpu.sync_copy(dst_hbm.at[w, pl.ds(ph * PHASE, PHASE)], dst_v)
        gather(0, 0)
        gather(1, 1)
        gather(2, 2)

        def body(j, carry2):
            for b in range(4):
                i = 4 * j + b
                gwait(i, b)
                sstart(i, b)
                bn = (b + 3) % 4

                @pl.when(i >= 1)
                def _():
                    swait(i - 1, bn)

                @pl.when(i + 3 < PHASE)
                def _():
                    gather(i + 3, bn)

            return carry2

        lax.fori_loop(0, PHASE // 4, body, 0)
        swait(PHASE - 1, (PHASE - 1) % 4)
        return carry

    lax.fori_loop(0, NPHASE, phase_body, 0)
    plsc.subcore_barrier()

    pltpu.sync_copy(acc.at[pl.ds(sid * TPN, TPN)],
                    out_hbm.at[cid, pl.ds(sid * TPN, TPN)])


@functools.cache
def _sc_scatter():
    return pl.kernel(
        _sc_body,
        out_type=jax.ShapeDtypeStruct((NUM_CORES, NPAD, C_OUT), jnp.float32),
        mesh=plsc.VectorSubcoreMesh(core_axis_name="c", subcore_axis_name="s",
                                    num_cores=NUM_CORES,
                                    num_subcores=NUM_SUBCORES),
        scratch_types=[
            pltpu.VMEM((PHASE, CHUNK), jnp.int32),
            pltpu.VMEM((PHASE, CHUNK), jnp.int32),
            pltpu.VMEM((CHUNK, C_OUT), jnp.float32),
            pltpu.VMEM((CHUNK, C_OUT), jnp.float32),
            pltpu.VMEM((CHUNK, C_OUT), jnp.float32),
            pltpu.VMEM((CHUNK, C_OUT), jnp.float32),
            pltpu.VMEM_SHARED((NPAD, C_OUT), jnp.float32),
        ] + [pltpu.SemaphoreType.DMA] * 8,
    )


def kernel(x, edge_index, edge_offset, W, gamma, beta):
    src = edge_index[0]
    dst = edge_index[1]

    H = pl.pallas_call(
        _h_body,
        grid=(KVOL,),
        in_specs=[
            pl.BlockSpec((N, C_IN), lambda k: (0, 0)),
            pl.BlockSpec((1, C_IN, C_OUT), lambda k: (k, 0, 0)),
        ],
        out_specs=pl.BlockSpec((1, N, C_OUT), lambda k: (k, 0, 0)),
        out_shape=jax.ShapeDtypeStruct((KVOL, N, C_OUT), jnp.float32),
    )(x, W)
    H2 = H.reshape(KVOL * N, C_OUT)

    gidx = pl.pallas_call(
        _gidx_body,
        out_shape=jax.ShapeDtypeStruct((E // C_OUT, C_OUT), jnp.int32),
    )(edge_offset.reshape(E // C_OUT, C_OUT), src.reshape(E // C_OUT, C_OUT))

    npadE = EPAD - E
    gidx_p = jnp.concatenate(
        [gidx.reshape(E),
         (jnp.arange(npadE, dtype=jnp.int32) * 997) % (KVOL * N)])
    dst_p = jnp.concatenate(
        [dst, N + jnp.arange(npadE, dtype=jnp.int32) % (NPAD - N)])
    parts = _sc_scatter()(
        H2,
        gidx_p.reshape(NUM_WORKERS, ROWS_PER_W, CHUNK),
        dst_p.reshape(NUM_WORKERS, ROWS_PER_W, CHUNK),
    )

    out = pl.pallas_call(
        _bn_body,
        out_shape=jax.ShapeDtypeStruct((N, C_OUT), jnp.float32),
    )(parts, gamma.reshape(1, C_OUT), beta.reshape(1, C_OUT))
    return out
